# Initial kernel scaffold; baseline (speedup 1.0000x reference)
import jax
import jax.numpy as jnp
from jax.experimental import pallas as pl


def kernel(x, edge_index, edge_attr, batch, We1, bE1, W1_1, b1_1, gam1, bet1, W2_1, b2_1, We2, bE2, W1_2, b1_2, gam2, bet2, W2_2, b2_2, We3, bE3, W1_3, b1_3, gam3, bet3, W2_3, b2_3, Wl1, bl1, Wl2, bl2):
    raise NotImplementedError("write your pallas kernel here")


# speedup vs baseline: 1.8858x; 1.8858x over previous
import functools

import jax
import jax.numpy as jnp
from jax import lax
from jax.experimental import pallas as pl
from jax.experimental.pallas import tpu as pltpu
from jax.experimental.pallas import tpu_sc as plsc

N = 10000
E = 320000
D = 128
HD = 64
D_EDGE = 16
N_GRAPHS = 128

NPAD = 10240
EP = 327680
EPT = EP // 16
RPT = EPT // 128
NJ = EPT // 1024
SUB = 512
ROWS_PER_TILE = NPAD // 16
ZROWS = 128


@functools.cache
def _make_edge_sc():
    mesh = plsc.VectorSubcoreMesh(core_axis_name="c", subcore_axis_name="s")

    @functools.partial(
        pl.kernel,
        mesh=mesh,
        compiler_params=pltpu.CompilerParams(use_tc_tiling_on_sc=False),
        out_type=jax.ShapeDtypeStruct((2, NPAD, HD), jnp.float32),
        scratch_types=[
            pltpu.VMEM((8, 128), jnp.int32),
            pltpu.VMEM((8, 128), jnp.int32),
            pltpu.VMEM((SUB, HD), jnp.float32),
            pltpu.VMEM((SUB, HD), jnp.float32),
            pltpu.VMEM((ZROWS, HD), jnp.float32),
            pltpu.VMEM_SHARED((NPAD, HD), jnp.float32),
            pltpu.SemaphoreType.DMA,
        ],
    )
    def edge_sc(x_hbm, src_hbm, dst_hbm, e_hbm, out_hbm,
                sidx, didx, xr, ev, zbuf, agg, sem):
        c = lax.axis_index("c")
        s = lax.axis_index("s")

        def zrow(r, carry):
            for k in range(HD // 16):
                zbuf[r, pl.ds(k * 16, 16)] = jnp.zeros((16,), jnp.float32)
            return carry
        lax.fori_loop(0, ZROWS, zrow, 0)
        for t in range(5):
            row0 = pl.multiple_of(s * ROWS_PER_TILE + t * ZROWS, 8)
            pltpu.sync_copy(zbuf, agg.at[pl.ds(row0, ZROWS), :])
        plsc.subcore_barrier()

        def body(j, carry):
            r0 = pl.multiple_of(s * RPT + j * 8, 8)
            pltpu.sync_copy(src_hbm.at[pl.ds(r0, 8)], sidx)
            pltpu.sync_copy(dst_hbm.at[pl.ds(r0, 8)], didx)
            for i in range(1024 // SUB):
                e0 = pl.multiple_of(r0 * 128 + i * SUB, 8)
                pltpu.sync_copy(e_hbm.at[c, pl.ds(e0, SUB), :], ev)
                cps = [pltpu.async_copy(
                           x_hbm.at[c].at[sidx.at[(SUB // 128) * i + u]],
                           xr.at[pl.ds(u * 128, 128)], sem)
                       for u in range(SUB // 128)]
                for cp in cps:
                    cp.wait()

                def crow(r, cc):
                    for k in range(HD // 16):
                        sl = pl.ds(k * 16, 16)
                        xr[r, sl] = jnp.maximum(xr[r, sl] + ev[r, sl], 0.0)
                    return cc
                lax.fori_loop(0, SUB, crow, 0)

                for u in range(SUB // 128):
                    pltpu.sync_copy(xr.at[pl.ds(u * 128, 128)],
                                    agg.at[didx.at[(SUB // 128) * i + u]],
                                    add=True)
            return carry
        lax.fori_loop(0, NJ, body, 0)
        plsc.subcore_barrier()

        for t in range(5):
            row0 = pl.multiple_of(s * ROWS_PER_TILE + t * ZROWS, 8)
            pltpu.sync_copy(agg.at[pl.ds(row0, ZROWS), :], zbuf)
            pltpu.sync_copy(zbuf, out_hbm.at[c, pl.ds(row0, ZROWS), :])

    return edge_sc


_EBLK = 2560


def _emb_body(attr, WeS, bES, e1, e2, e3):
    prod = jnp.dot(attr[...], WeS[...],
                   preferred_element_type=jnp.float32) + bES[...]
    for l, e_ref in enumerate((e1, e2, e3)):
        e_ref[0] = prod[:, l * D:l * D + HD]
        e_ref[1] = prod[:, l * D + HD:(l + 1) * D]


def _edge_embed(attr_p, WeS, bES):
    grid = (EP // _EBLK,)
    return pl.pallas_call(
        _emb_body,
        grid=grid,
        in_specs=[
            pl.BlockSpec((_EBLK, D_EDGE), lambda i: (i, 0)),
            pl.BlockSpec((D_EDGE, 3 * D), lambda i: (0, 0)),
            pl.BlockSpec((1, 3 * D), lambda i: (0, 0)),
        ],
        out_specs=[
            pl.BlockSpec((2, _EBLK, HD), lambda i: (0, i, 0)),
            pl.BlockSpec((2, _EBLK, HD), lambda i: (0, i, 0)),
            pl.BlockSpec((2, _EBLK, HD), lambda i: (0, i, 0)),
        ],
        out_shape=[jax.ShapeDtypeStruct((2, EP, HD), jnp.float32)] * 3,
    )(attr_p, WeS, bES)


def _node_body(x, a2, b2d, W1, b1, gam, bet, W2, b2, h_out, hs_out, p_out):
    agg = jnp.concatenate([a2[0, 0:N, :], a2[1, 0:N, :]], axis=1)
    h = x[...] + agg
    h = jnp.dot(h, W1[...], preferred_element_type=jnp.float32) + b1[...]
    mu = jnp.mean(h, axis=0, keepdims=True)
    var = jnp.mean((h - mu) ** 2, axis=0, keepdims=True)
    h = (h - mu) * lax.rsqrt(var + 1e-5) * gam[...] + bet[...]
    h = jnp.maximum(h, 0.0)
    h = jnp.dot(h, W2[...], preferred_element_type=jnp.float32) + b2[...]
    h = jnp.maximum(h, 0.0)
    h_out[...] = h
    hs_out[0] = h[:, 0:HD]
    hs_out[1] = h[:, HD:D]
    oh = (lax.broadcasted_iota(jnp.int32, (N_GRAPHS, N), 0)
          == b2d[...]).astype(jnp.float32)
    p_out[...] = jnp.dot(oh, h, preferred_element_type=jnp.float32)


def _node_mlp(x, agg2, b2d, W1, b1, gam, bet, W2, b2):
    return pl.pallas_call(
        _node_body,
        out_shape=[
            jax.ShapeDtypeStruct((N, D), jnp.float32),
            jax.ShapeDtypeStruct((2, N, HD), jnp.float32),
            jax.ShapeDtypeStruct((N_GRAPHS, D), jnp.float32),
        ],
    )(x, agg2, b2d, W1, b1, gam, bet, W2, b2)


def _head_body(p1, p2, p3, Wl1, bl1, Wl2, bl2, out):
    h = jnp.concatenate([p1[...], p2[...], p3[...]], axis=1)
    h = jnp.dot(h, Wl1[...], preferred_element_type=jnp.float32) + bl1[...]
    h = jnp.maximum(h, 0.0)
    out[...] = jnp.dot(h, Wl2[...], preferred_element_type=jnp.float32) + bl2[...]


def _head(p1, p2, p3, Wl1, bl1, Wl2, bl2):
    return pl.pallas_call(
        _head_body,
        out_shape=jax.ShapeDtypeStruct((N_GRAPHS, 1), jnp.float32),
    )(p1, p2, p3, Wl1, bl1, Wl2, bl2)


def kernel(x, edge_index, edge_attr, batch,
           We1, bE1, W1_1, b1_1, gam1, bet1, W2_1, b2_1,
           We2, bE2, W1_2, b1_2, gam2, bet2, W2_2, b2_2,
           We3, bE3, W1_3, b1_3, gam3, bet3, W2_3, b2_3,
           Wl1, bl1, Wl2, bl2):
    pad = EP - E
    src2d = jnp.concatenate(
        [edge_index[0], jnp.zeros((pad,), jnp.int32)]).reshape(EP // 128, 128)
    dst2d = jnp.concatenate(
        [edge_index[1], jnp.full((pad,), NPAD - 1, jnp.int32)]).reshape(
            EP // 128, 128)
    attr_p = jnp.concatenate(
        [edge_attr, jnp.zeros((pad, D_EDGE), jnp.float32)], axis=0)
    WeS = jnp.concatenate([We1, We2, We3], axis=1)
    bES = jnp.concatenate([bE1, bE2, bE3]).reshape(1, 3 * D)
    e1, e2, e3 = _edge_embed(attr_p, WeS, bES)
    b2d = batch.reshape(1, N)

    h = x
    hs = jnp.stack([x[:, 0:HD], x[:, HD:D]])
    ps = []
    for (e_l, W1, b1, gam, bet, W2, b2) in (
            (e1, W1_1, b1_1, gam1, bet1, W2_1, b2_1),
            (e2, W1_2, b1_2, gam2, bet2, W2_2, b2_2),
            (e3, W1_3, b1_3, gam3, bet3, W2_3, b2_3)):
        agg2 = _make_edge_sc()(hs, src2d, dst2d, e_l)
        h, hs, p = _node_mlp(h, agg2, b2d,
                             W1, b1.reshape(1, D), gam.reshape(1, D),
                             bet.reshape(1, D), W2, b2.reshape(1, D))
        ps.append(p)

    return _head(ps[0], ps[1], ps[2],
                 Wl1, bl1.reshape(1, 3 * D), Wl2, bl2.reshape(1, 1))


# speedup vs baseline: 1.9073x; 1.0114x over previous
import functools

import jax
import jax.numpy as jnp
from jax import lax
from jax.experimental import pallas as pl
from jax.experimental.pallas import tpu as pltpu
from jax.experimental.pallas import tpu_sc as plsc

N = 10000
E = 320000
D = 128
HD = 64
D_EDGE = 16
N_GRAPHS = 128

NPAD = 10240
EP = 327680
EPT = EP // 16
RPT = EPT // 128
NJ = EPT // 1024
SUB = 512
ROWS_PER_TILE = NPAD // 16
ZROWS = 128


@functools.cache
def _make_edge_sc():
    mesh = plsc.VectorSubcoreMesh(core_axis_name="c", subcore_axis_name="s")

    @functools.partial(
        pl.kernel,
        mesh=mesh,
        compiler_params=pltpu.CompilerParams(use_tc_tiling_on_sc=False),
        out_type=jax.ShapeDtypeStruct((2, NPAD, HD), jnp.float32),
        scratch_types=[
            pltpu.VMEM((8, 128), jnp.int32),
            pltpu.VMEM((8, 128), jnp.int32),
            pltpu.VMEM((SUB, HD), jnp.float32),
            pltpu.VMEM((SUB, HD), jnp.float32),
            pltpu.VMEM((ZROWS, HD), jnp.float32),
            pltpu.VMEM_SHARED((NPAD, HD), jnp.float32),
            pltpu.SemaphoreType.DMA,
        ],
    )
    def edge_sc(x_hbm, src_hbm, dst_hbm, e_hbm, out_hbm,
                sidx, didx, xr, ev, zbuf, agg, sem):
        c = lax.axis_index("c")
        s = lax.axis_index("s")

        def zrow(r, carry):
            for k in range(HD // 16):
                zbuf[r, pl.ds(k * 16, 16)] = jnp.zeros((16,), jnp.float32)
            return carry
        lax.fori_loop(0, ZROWS, zrow, 0)
        for t in range(5):
            row0 = pl.multiple_of(s * ROWS_PER_TILE + t * ZROWS, 8)
            pltpu.sync_copy(zbuf, agg.at[pl.ds(row0, ZROWS), :])
        plsc.subcore_barrier()

        def body(j, carry):
            r0 = pl.multiple_of(s * RPT + j * 8, 8)
            pltpu.sync_copy(src_hbm.at[pl.ds(r0, 8)], sidx)
            pltpu.sync_copy(dst_hbm.at[pl.ds(r0, 8)], didx)
            for i in range(1024 // SUB):
                e0 = pl.multiple_of(r0 * 128 + i * SUB, 8)
                pltpu.sync_copy(e_hbm.at[c, pl.ds(e0, SUB), :], ev)
                cps = [pltpu.async_copy(
                           x_hbm.at[c].at[sidx.at[(SUB // 128) * i + u]],
                           xr.at[pl.ds(u * 128, 128)], sem)
                       for u in range(SUB // 128)]
                for cp in cps:
                    cp.wait()

                def crow(r8, cc):
                    r0c = r8 * 8
                    for u in range(8):
                        for k in range(HD // 16):
                            sl = pl.ds(k * 16, 16)
                            xr[r0c + u, sl] = jnp.maximum(
                                xr[r0c + u, sl] + ev[r0c + u, sl], 0.0)
                    return cc
                lax.fori_loop(0, SUB // 8, crow, 0)

                for u in range(SUB // 128):
                    pltpu.sync_copy(xr.at[pl.ds(u * 128, 128)],
                                    agg.at[didx.at[(SUB // 128) * i + u]],
                                    add=True)
            return carry
        lax.fori_loop(0, NJ, body, 0)
        plsc.subcore_barrier()

        for t in range(5):
            row0 = pl.multiple_of(s * ROWS_PER_TILE + t * ZROWS, 8)
            pltpu.sync_copy(agg.at[pl.ds(row0, ZROWS), :], zbuf)
            pltpu.sync_copy(zbuf, out_hbm.at[c, pl.ds(row0, ZROWS), :])

    return edge_sc


_EBLK = 2560


def _emb_body(attr, WeS, bES, e1, e2, e3):
    prod = jnp.dot(attr[...], WeS[...],
                   preferred_element_type=jnp.float32) + bES[...]
    for l, e_ref in enumerate((e1, e2, e3)):
        e_ref[0] = prod[:, l * D:l * D + HD]
        e_ref[1] = prod[:, l * D + HD:(l + 1) * D]


def _edge_embed(attr_p, WeS, bES):
    grid = (EP // _EBLK,)
    return pl.pallas_call(
        _emb_body,
        grid=grid,
        in_specs=[
            pl.BlockSpec((_EBLK, D_EDGE), lambda i: (i, 0)),
            pl.BlockSpec((D_EDGE, 3 * D), lambda i: (0, 0)),
            pl.BlockSpec((1, 3 * D), lambda i: (0, 0)),
        ],
        out_specs=[
            pl.BlockSpec((2, _EBLK, HD), lambda i: (0, i, 0)),
            pl.BlockSpec((2, _EBLK, HD), lambda i: (0, i, 0)),
            pl.BlockSpec((2, _EBLK, HD), lambda i: (0, i, 0)),
        ],
        out_shape=[jax.ShapeDtypeStruct((2, EP, HD), jnp.float32)] * 3,
    )(attr_p, WeS, bES)


def _node_body(x, a2, b2d, W1, b1, gam, bet, W2, b2, h_out, hs_out, p_out):
    agg = jnp.concatenate([a2[0, 0:N, :], a2[1, 0:N, :]], axis=1)
    h = x[...] + agg
    h = jnp.dot(h, W1[...], preferred_element_type=jnp.float32) + b1[...]
    mu = jnp.mean(h, axis=0, keepdims=True)
    var = jnp.mean((h - mu) ** 2, axis=0, keepdims=True)
    h = (h - mu) * lax.rsqrt(var + 1e-5) * gam[...] + bet[...]
    h = jnp.maximum(h, 0.0)
    h = jnp.dot(h, W2[...], preferred_element_type=jnp.float32) + b2[...]
    h = jnp.maximum(h, 0.0)
    h_out[...] = h
    hs_out[0] = h[:, 0:HD]
    hs_out[1] = h[:, HD:D]
    oh = (lax.broadcasted_iota(jnp.int32, (N_GRAPHS, N), 0)
          == b2d[...]).astype(jnp.float32)
    p_out[...] = jnp.dot(oh, h, preferred_element_type=jnp.float32)


def _node_mlp(x, agg2, b2d, W1, b1, gam, bet, W2, b2):
    return pl.pallas_call(
        _node_body,
        out_shape=[
            jax.ShapeDtypeStruct((N, D), jnp.float32),
            jax.ShapeDtypeStruct((2, N, HD), jnp.float32),
            jax.ShapeDtypeStruct((N_GRAPHS, D), jnp.float32),
        ],
    )(x, agg2, b2d, W1, b1, gam, bet, W2, b2)


def _head_body(p1, p2, p3, Wl1, bl1, Wl2, bl2, out):
    h = jnp.concatenate([p1[...], p2[...], p3[...]], axis=1)
    h = jnp.dot(h, Wl1[...], preferred_element_type=jnp.float32) + bl1[...]
    h = jnp.maximum(h, 0.0)
    out[...] = jnp.dot(h, Wl2[...], preferred_element_type=jnp.float32) + bl2[...]


def _head(p1, p2, p3, Wl1, bl1, Wl2, bl2):
    return pl.pallas_call(
        _head_body,
        out_shape=jax.ShapeDtypeStruct((N_GRAPHS, 1), jnp.float32),
    )(p1, p2, p3, Wl1, bl1, Wl2, bl2)


def kernel(x, edge_index, edge_attr, batch,
           We1, bE1, W1_1, b1_1, gam1, bet1, W2_1, b2_1,
           We2, bE2, W1_2, b1_2, gam2, bet2, W2_2, b2_2,
           We3, bE3, W1_3, b1_3, gam3, bet3, W2_3, b2_3,
           Wl1, bl1, Wl2, bl2):
    pad = EP - E
    src2d = jnp.concatenate(
        [edge_index[0], jnp.zeros((pad,), jnp.int32)]).reshape(EP // 128, 128)
    dst2d = jnp.concatenate(
        [edge_index[1], jnp.full((pad,), NPAD - 1, jnp.int32)]).reshape(
            EP // 128, 128)
    attr_p = jnp.concatenate(
        [edge_attr, jnp.zeros((pad, D_EDGE), jnp.float32)], axis=0)
    WeS = jnp.concatenate([We1, We2, We3], axis=1)
    bES = jnp.concatenate([bE1, bE2, bE3]).reshape(1, 3 * D)
    e1, e2, e3 = _edge_embed(attr_p, WeS, bES)
    b2d = batch.reshape(1, N)

    h = x
    hs = jnp.stack([x[:, 0:HD], x[:, HD:D]])
    ps = []
    for (e_l, W1, b1, gam, bet, W2, b2) in (
            (e1, W1_1, b1_1, gam1, bet1, W2_1, b2_1),
            (e2, W1_2, b1_2, gam2, bet2, W2_2, b2_2),
            (e3, W1_3, b1_3, gam3, bet3, W2_3, b2_3)):
        agg2 = _make_edge_sc()(hs, src2d, dst2d, e_l)
        h, hs, p = _node_mlp(h, agg2, b2d,
                             W1, b1.reshape(1, D), gam.reshape(1, D),
                             bet.reshape(1, D), W2, b2.reshape(1, D))
        ps.append(p)

    return _head(ps[0], ps[1], ps[2],
                 Wl1, bl1.reshape(1, 3 * D), Wl2, bl2.reshape(1, 1))
